# Initial kernel scaffold; baseline (speedup 1.0000x reference)
import jax
import jax.numpy as jnp
from jax.experimental import pallas as pl


def kernel(x, edge_index, Wl1, Wr1, b1, Wl2, Wr2, b2, Wl3, Wr3, b3, Wl4, Wr4, b4):
    raise NotImplementedError("write your pallas kernel here")


# speedup vs baseline: 5.5989x; 5.5989x over previous
import jax
import jax.numpy as jnp
from jax import lax
from jax.experimental import pallas as pl
from jax.experimental.pallas import tpu as pltpu
from jax.experimental.pallas import tpu_sc as plsc

N_NODES = 10000
NP = 10240
DIM = 128
HD = DIM // 2
NC, NS = 2, 16
NW = NC * NS
CH = 128
ROWS_PER_TILE = NP // NS
SLICES_PER_TILE = ROWS_PER_TILE // CH


def _sc_mesh():
    return plsc.VectorSubcoreMesh(
        core_axis_name="c", subcore_axis_name="s",
        num_cores=NC, num_subcores=NS)


def _make_agg(e_pad, d, split_features, with_count):
    if split_features:
        chunks_per_worker = e_pad // (NS * CH)
    else:
        chunks_per_worker = e_pad // (NW * CH)
    n_out = 2 if with_count else 1

    def body(src_hbm, dst_hbm, x_hbm, *refs):
        if with_count:
            out_hbm, cnt_hbm = refs[0], refs[1]
        else:
            out_hbm = refs[0]
        scratch = refs[n_out:]
        if with_count:
            sidx, didx, rows, z16, ones, acc_s, cnt_s, sem = scratch
        else:
            sidx, didx, rows, z16, acc_s, sem = scratch

        cid = lax.axis_index("c")
        sid = lax.axis_index("s")
        wid = sid * NC + cid

        def fill(r, _):
            for k in range(d // 16):
                rows[r, pl.ds(k * 16, 16)] = jnp.zeros((16,), jnp.float32)
            z16[r] = jnp.zeros((16,), jnp.float32)
            if with_count:
                ones[r] = jnp.ones((16,), jnp.float32)
            return 0
        lax.fori_loop(0, CH, fill, 0)

        for j in range(SLICES_PER_TILE):
            r0 = sid * ROWS_PER_TILE + j * CH
            pltpu.sync_copy(rows, acc_s.at[pl.ds(r0, CH)])
            if with_count:
                pltpu.sync_copy(z16, cnt_s.at[pl.ds(r0, CH)])
        plsc.subcore_barrier()

        base0 = (sid if split_features else wid) * chunks_per_worker

        def chunk(i, _):
            e0 = (base0 + i) * CH
            pltpu.sync_copy(src_hbm.at[pl.ds(e0, CH)], sidx)
            pltpu.sync_copy(dst_hbm.at[pl.ds(e0, CH)], didx)
            if split_features:
                off = cid * NP
                for k in range(CH // 16):
                    sidx[pl.ds(k * 16, 16)] = sidx[pl.ds(k * 16, 16)] + off
            pltpu.async_copy(x_hbm.at[sidx], rows, sem).wait()
            pltpu.sync_copy(rows, acc_s.at[didx], add=True)
            if with_count:
                @pl.when(cid == 0)
                def _():
                    pltpu.sync_copy(ones, cnt_s.at[didx], add=True)
            return 0
        lax.fori_loop(0, chunks_per_worker, chunk, 0)
        plsc.subcore_barrier()

        for j in range(SLICES_PER_TILE):
            r0 = sid * ROWS_PER_TILE + j * CH
            pltpu.sync_copy(acc_s.at[pl.ds(r0, CH)],
                            out_hbm.at[pl.ds(cid * NP + r0, CH)])
            if with_count:
                @pl.when(cid == 0)
                def _():
                    pltpu.sync_copy(cnt_s.at[pl.ds(r0, CH)],
                                    cnt_hbm.at[pl.ds(r0, CH)])

    out_type = [jax.ShapeDtypeStruct((NC * NP, d), jnp.float32)]
    scratch = [
        pltpu.VMEM((CH,), jnp.int32),
        pltpu.VMEM((CH,), jnp.int32),
        pltpu.VMEM((CH, d), jnp.float32),
        pltpu.VMEM((CH, 16), jnp.float32),
    ]
    if with_count:
        out_type.append(jax.ShapeDtypeStruct((NP, 16), jnp.float32))
        scratch.append(pltpu.VMEM((CH, 16), jnp.float32))
    scratch.append(pltpu.VMEM_SHARED((NP, d), jnp.float32))
    if with_count:
        scratch.append(pltpu.VMEM_SHARED((NP, 16), jnp.float32))
    scratch.append(pltpu.SemaphoreType.DMA)

    return pl.kernel(body, out_type=out_type, mesh=_sc_mesh(),
                     scratch_types=scratch,
                     compiler_params=pltpu.CompilerParams(
                         use_tc_tiling_on_sc=False))


NB = 1024
GRID = NP // NB


def _invd(cnt_ref):
    deg = cnt_ref[:, 0:1]
    return 1.0 / jnp.maximum(deg, 1.0)


def _tc_layer12(p_ref, cnt_ref, h_ref, wl_ref, wr_ref, b_ref, o_ref):
    agg = jnp.concatenate([p_ref[0], p_ref[1]], axis=1) * _invd(cnt_ref)
    h = (jnp.dot(agg, wl_ref[...], preferred_element_type=jnp.float32)
         + jnp.dot(h_ref[...], wr_ref[...], preferred_element_type=jnp.float32)
         + b_ref[...])
    o_ref[...] = jnp.maximum(h, 0.0)


def _tc_layer2b(p_ref, cnt_ref, h_ref, wl_ref, wr_ref, b_ref, w3l_ref, w3r_ref,
                y_ref, z_ref):
    agg = jnp.concatenate([p_ref[0], p_ref[1]], axis=1) * _invd(cnt_ref)
    h2 = (jnp.dot(agg, wl_ref[...], preferred_element_type=jnp.float32)
          + jnp.dot(h_ref[...], wr_ref[...], preferred_element_type=jnp.float32)
          + b_ref[...])
    h2 = jnp.maximum(h2, 0.0)
    y_ref[...] = jnp.dot(h2, w3l_ref[...], preferred_element_type=jnp.float32)
    z_ref[...] = jnp.dot(h2, w3r_ref[...], preferred_element_type=jnp.float32)


def _tc_layer3(a_ref, cnt_ref, z_ref, b_ref, o_ref):
    h3 = ((a_ref[0, :, 0:1] + a_ref[1, :, 0:1]) * _invd(cnt_ref)
          + z_ref[:, 0:1] + b_ref[...])
    cols = lax.broadcasted_iota(jnp.int32, (NB, 16), 1)
    o_ref[...] = jnp.where(cols == 0, h3, 0.0)


def _tc_layer4(a_ref, cnt_ref, h3_ref, wl_ref, wr_ref, b_ref, o_ref):
    a4n = (a_ref[0, :, 0:1] + a_ref[1, :, 0:1]) * _invd(cnt_ref)
    logits = (a4n * wl_ref[...] + h3_ref[:, 0:1] * wr_ref[...] + b_ref[...])
    m = jnp.max(logits, axis=1, keepdims=True)
    sh = logits - m
    lse = jnp.log(jnp.sum(jnp.exp(sh), axis=1, keepdims=True))
    o_ref[...] = sh - lse


def _spec_ph():
    return pl.BlockSpec((2, NB, HD), lambda i: (0, i, 0))


def _spec_16x2():
    return pl.BlockSpec((2, NB, 16), lambda i: (0, i, 0))


def _spec_rows(w):
    return pl.BlockSpec((NB, w), lambda i: (i, 0))


def _spec_full(shape):
    return pl.BlockSpec(shape, lambda i: tuple(0 for _ in shape))


def kernel(x, edge_index, Wl1, Wr1, b1, Wl2, Wr2, b2, Wl3, Wr3, b3,
           Wl4, Wr4, b4):
    x = x.astype(jnp.float32)
    src = edge_index[0].astype(jnp.int32)
    dst = edge_index[1].astype(jnp.int32)
    e = src.shape[0]
    quantum = NW * CH
    e_pad = -(-e // quantum) * quantum
    if e_pad != e:
        src = jnp.concatenate([src, jnp.zeros((e_pad - e,), jnp.int32)])
        dst = jnp.concatenate(
            [dst, jnp.full((e_pad - e,), N_NODES, jnp.int32)])
    xp = jnp.pad(x, ((0, NP - x.shape[0]), (0, 0)))

    agg_wide_cnt = _make_agg(e_pad, HD, True, True)
    agg_wide = _make_agg(e_pad, HD, True, False)
    agg_16 = _make_agg(e_pad, 16, False, False)

    def halves(v):
        return v.reshape(NP, 2, HD).transpose(1, 0, 2).reshape(2 * NP, HD)

    p1, cnt = agg_wide_cnt(src, dst, halves(xp))
    h1 = pl.pallas_call(
        _tc_layer12,
        grid=(GRID,),
        in_specs=[_spec_ph(), _spec_rows(16), _spec_rows(DIM),
                  _spec_full((DIM, DIM)), _spec_full((DIM, DIM)),
                  _spec_full((1, DIM))],
        out_specs=_spec_rows(DIM),
        out_shape=jax.ShapeDtypeStruct((NP, DIM), jnp.float32),
    )(p1.reshape(2, NP, HD), cnt, xp, Wl1, Wr1, b1.reshape(1, DIM))

    p2 = agg_wide(src, dst, halves(h1))[0]
    w3l = jnp.pad(Wl3, ((0, 0), (0, 15)))
    w3r = jnp.pad(Wr3, ((0, 0), (0, 15)))
    y3w, z3w = pl.pallas_call(
        _tc_layer2b,
        grid=(GRID,),
        in_specs=[_spec_ph(), _spec_rows(16), _spec_rows(DIM),
                  _spec_full((DIM, DIM)), _spec_full((DIM, DIM)),
                  _spec_full((1, DIM)), _spec_full((DIM, 16)),
                  _spec_full((DIM, 16))],
        out_specs=[_spec_rows(16), _spec_rows(16)],
        out_shape=[jax.ShapeDtypeStruct((NP, 16), jnp.float32),
                   jax.ShapeDtypeStruct((NP, 16), jnp.float32)],
    )(p2.reshape(2, NP, HD), cnt, h1, Wl2, Wr2, b2.reshape(1, DIM), w3l, w3r)

    a3 = agg_16(src, dst, y3w)[0]
    h3w = pl.pallas_call(
        _tc_layer3,
        grid=(GRID,),
        in_specs=[_spec_16x2(), _spec_rows(16), _spec_rows(16),
                  _spec_full((1, 1))],
        out_specs=_spec_rows(16),
        out_shape=jax.ShapeDtypeStruct((NP, 16), jnp.float32),
    )(a3.reshape(2, NP, 16), cnt, z3w, b3.reshape(1, 1))

    a4 = agg_16(src, dst, h3w)[0]
    out = pl.pallas_call(
        _tc_layer4,
        grid=(GRID,),
        in_specs=[_spec_16x2(), _spec_rows(16), _spec_rows(16),
                  _spec_full((1, 16)), _spec_full((1, 16)),
                  _spec_full((1, 16))],
        out_specs=_spec_rows(16),
        out_shape=jax.ShapeDtypeStruct((NP, 16), jnp.float32),
    )(a4.reshape(2, NP, 16), cnt, h3w, Wl4, Wr4, b4.reshape(1, 16))

    n = x.shape[0]
    return (out[:n], h3w[:n, 0])


# speedup vs baseline: 8.1367x; 1.4533x over previous
import jax
import jax.numpy as jnp
from jax import lax
from jax.experimental import pallas as pl
from jax.experimental.pallas import tpu as pltpu
from jax.experimental.pallas import tpu_sc as plsc

N_NODES = 10000
NP = 10240
DIM = 128
HD = DIM // 2
NC, NS = 2, 16
NW = NC * NS
CH = 128
ROWS_PER_TILE = NP // NS
SLICES_PER_TILE = ROWS_PER_TILE // CH


def _sc_mesh():
    return plsc.VectorSubcoreMesh(
        core_axis_name="c", subcore_axis_name="s",
        num_cores=NC, num_subcores=NS)


NBUF = 4


def _make_agg(e_pad, d, split_features, with_count):
    if split_features:
        chunks_per_worker = e_pad // (NS * CH)
    else:
        chunks_per_worker = e_pad // (NW * CH)
    nsuper = chunks_per_worker // NBUF
    n_out = 2 if with_count else 1

    def body(src_hbm, dst_hbm, x_hbm, *refs):
        if with_count:
            out_hbm, cnt_hbm = refs[0], refs[1]
        else:
            out_hbm = refs[0]
        scratch = refs[n_out:]
        if with_count:
            sidx, didx, rows, z16, ones, acc_s, cnt_s, gsem, ssem, csem = scratch
        else:
            sidx, didx, rows, z16, acc_s, gsem, ssem = scratch

        cid = lax.axis_index("c")
        sid = lax.axis_index("s")
        wid = sid * NC + cid

        def fill(r, _):
            for k in range(d // 16):
                rows[0, r, pl.ds(k * 16, 16)] = jnp.zeros((16,), jnp.float32)
            z16[r] = jnp.zeros((16,), jnp.float32)
            if with_count:
                ones[r] = jnp.ones((16,), jnp.float32)
            return 0
        lax.fori_loop(0, CH, fill, 0)

        for j in range(SLICES_PER_TILE):
            r0 = sid * ROWS_PER_TILE + j * CH
            pltpu.sync_copy(rows.at[0], acc_s.at[pl.ds(r0, CH)])
            if with_count:
                pltpu.sync_copy(z16, cnt_s.at[pl.ds(r0, CH)])
        plsc.subcore_barrier()

        base0 = (sid if split_features else wid) * chunks_per_worker

        def load_idx(s, parity):
            row0 = base0 + s * NBUF
            pltpu.sync_copy(src_hbm.at[pl.ds(row0, NBUF)], sidx.at[parity])
            pltpu.sync_copy(dst_hbm.at[pl.ds(row0, NBUF)], didx.at[parity])
            if split_features:
                off = cid * NP
                for b in range(NBUF):
                    for k in range(CH // 16):
                        sidx[parity, b, pl.ds(k * 16, 16)] = (
                            sidx[parity, b, pl.ds(k * 16, 16)] + off)

        def gather(parity, b):
            return pltpu.make_async_copy(
                x_hbm.at[sidx.at[parity, b]], rows.at[b], gsem.at[b])

        def scatter(parity, b):
            return pltpu.make_async_copy(
                rows.at[b], acc_s.at[didx.at[parity, b]], ssem.at[b])

        def cscatter(parity, b):
            return pltpu.make_async_copy(
                ones, cnt_s.at[didx.at[parity, b]], csem.at[b])

        load_idx(0, 0)
        for b in range(NBUF):
            gather(0, b).start()

        def superstep(s, _):
            p = lax.rem(s, 2)
            np_ = 1 - p

            @pl.when(s < nsuper - 1)
            def _():
                load_idx(s + 1, np_)

            for b in range(NBUF):
                gather(p, b).wait()
                scatter(p, b).start(add=True)
                if with_count:
                    @pl.when(cid == 0)
                    def _():
                        cscatter(p, b).start(add=True)
            for b in range(NBUF):
                scatter(p, b).wait()
                if with_count:
                    @pl.when(cid == 0)
                    def _():
                        cscatter(p, b).wait()

                @pl.when(s < nsuper - 1)
                def _():
                    gather(np_, b).start()
            return 0
        lax.fori_loop(0, nsuper, superstep, 0)
        plsc.subcore_barrier()

        for j in range(SLICES_PER_TILE):
            r0 = sid * ROWS_PER_TILE + j * CH
            pltpu.sync_copy(acc_s.at[pl.ds(r0, CH)],
                            out_hbm.at[pl.ds(cid * NP + r0, CH)])
            if with_count:
                @pl.when(cid == 0)
                def _():
                    pltpu.sync_copy(cnt_s.at[pl.ds(r0, CH)],
                                    cnt_hbm.at[pl.ds(r0, CH)])

    out_type = [jax.ShapeDtypeStruct((NC * NP, d), jnp.float32)]
    scratch = [
        pltpu.VMEM((2, NBUF, CH), jnp.int32),
        pltpu.VMEM((2, NBUF, CH), jnp.int32),
        pltpu.VMEM((NBUF, CH, d), jnp.float32),
        pltpu.VMEM((CH, 16), jnp.float32),
    ]
    if with_count:
        out_type.append(jax.ShapeDtypeStruct((NP, 16), jnp.float32))
        scratch.append(pltpu.VMEM((CH, 16), jnp.float32))
    scratch.append(pltpu.VMEM_SHARED((NP, d), jnp.float32))
    if with_count:
        scratch.append(pltpu.VMEM_SHARED((NP, 16), jnp.float32))
    scratch.append(pltpu.SemaphoreType.DMA((NBUF,)))
    scratch.append(pltpu.SemaphoreType.DMA((NBUF,)))
    if with_count:
        scratch.append(pltpu.SemaphoreType.DMA((NBUF,)))

    return pl.kernel(body, out_type=out_type, mesh=_sc_mesh(),
                     scratch_types=scratch,
                     compiler_params=pltpu.CompilerParams(
                         use_tc_tiling_on_sc=False))


NB = 1024
GRID = NP // NB


def _invd(cnt_ref):
    deg = cnt_ref[:, 0:1]
    return 1.0 / jnp.maximum(deg, 1.0)


def _tc_layer12(p_ref, cnt_ref, h_ref, wl_ref, wr_ref, b_ref, o_ref):
    agg = jnp.concatenate([p_ref[0], p_ref[1]], axis=1) * _invd(cnt_ref)
    h = (jnp.dot(agg, wl_ref[...], preferred_element_type=jnp.float32)
         + jnp.dot(h_ref[...], wr_ref[...], preferred_element_type=jnp.float32)
         + b_ref[...])
    o_ref[...] = jnp.maximum(h, 0.0)


def _tc_layer2b(p_ref, cnt_ref, h_ref, wl_ref, wr_ref, b_ref, w3l_ref, w3r_ref,
                y_ref, z_ref):
    agg = jnp.concatenate([p_ref[0], p_ref[1]], axis=1) * _invd(cnt_ref)
    h2 = (jnp.dot(agg, wl_ref[...], preferred_element_type=jnp.float32)
          + jnp.dot(h_ref[...], wr_ref[...], preferred_element_type=jnp.float32)
          + b_ref[...])
    h2 = jnp.maximum(h2, 0.0)
    y_ref[...] = jnp.dot(h2, w3l_ref[...], preferred_element_type=jnp.float32)
    z_ref[...] = jnp.dot(h2, w3r_ref[...], preferred_element_type=jnp.float32)


def _tc_layer3(a_ref, cnt_ref, z_ref, b_ref, o_ref):
    h3 = ((a_ref[0, :, 0:1] + a_ref[1, :, 0:1]) * _invd(cnt_ref)
          + z_ref[:, 0:1] + b_ref[...])
    cols = lax.broadcasted_iota(jnp.int32, (NB, 16), 1)
    o_ref[...] = jnp.where(cols == 0, h3, 0.0)


def _tc_layer4(a_ref, cnt_ref, h3_ref, wl_ref, wr_ref, b_ref, o_ref):
    a4n = (a_ref[0, :, 0:1] + a_ref[1, :, 0:1]) * _invd(cnt_ref)
    logits = (a4n * wl_ref[...] + h3_ref[:, 0:1] * wr_ref[...] + b_ref[...])
    m = jnp.max(logits, axis=1, keepdims=True)
    sh = logits - m
    lse = jnp.log(jnp.sum(jnp.exp(sh), axis=1, keepdims=True))
    o_ref[...] = sh - lse


def _spec_ph():
    return pl.BlockSpec((2, NB, HD), lambda i: (0, i, 0))


def _spec_16x2():
    return pl.BlockSpec((2, NB, 16), lambda i: (0, i, 0))


def _spec_rows(w):
    return pl.BlockSpec((NB, w), lambda i: (i, 0))


def _spec_full(shape):
    return pl.BlockSpec(shape, lambda i: tuple(0 for _ in shape))


def kernel(x, edge_index, Wl1, Wr1, b1, Wl2, Wr2, b2, Wl3, Wr3, b3,
           Wl4, Wr4, b4):
    x = x.astype(jnp.float32)
    src = edge_index[0].astype(jnp.int32)
    dst = edge_index[1].astype(jnp.int32)
    e = src.shape[0]
    quantum = NW * CH * NBUF
    e_pad = -(-e // quantum) * quantum
    if e_pad != e:
        src = jnp.concatenate([src, jnp.zeros((e_pad - e,), jnp.int32)])
        dst = jnp.concatenate(
            [dst, jnp.full((e_pad - e,), N_NODES, jnp.int32)])
    src = src.reshape(e_pad // CH, CH)
    dst = dst.reshape(e_pad // CH, CH)
    xp = jnp.pad(x, ((0, NP - x.shape[0]), (0, 0)))

    agg_wide_cnt = _make_agg(e_pad, HD, True, True)
    agg_wide = _make_agg(e_pad, HD, True, False)
    agg_16 = _make_agg(e_pad, 16, False, False)

    def halves(v):
        return v.reshape(NP, 2, HD).transpose(1, 0, 2).reshape(2 * NP, HD)

    p1, cnt = agg_wide_cnt(src, dst, halves(xp))
    h1 = pl.pallas_call(
        _tc_layer12,
        grid=(GRID,),
        in_specs=[_spec_ph(), _spec_rows(16), _spec_rows(DIM),
                  _spec_full((DIM, DIM)), _spec_full((DIM, DIM)),
                  _spec_full((1, DIM))],
        out_specs=_spec_rows(DIM),
        out_shape=jax.ShapeDtypeStruct((NP, DIM), jnp.float32),
    )(p1.reshape(2, NP, HD), cnt, xp, Wl1, Wr1, b1.reshape(1, DIM))

    p2 = agg_wide(src, dst, halves(h1))[0]
    w3l = jnp.pad(Wl3, ((0, 0), (0, 15)))
    w3r = jnp.pad(Wr3, ((0, 0), (0, 15)))
    y3w, z3w = pl.pallas_call(
        _tc_layer2b,
        grid=(GRID,),
        in_specs=[_spec_ph(), _spec_rows(16), _spec_rows(DIM),
                  _spec_full((DIM, DIM)), _spec_full((DIM, DIM)),
                  _spec_full((1, DIM)), _spec_full((DIM, 16)),
                  _spec_full((DIM, 16))],
        out_specs=[_spec_rows(16), _spec_rows(16)],
        out_shape=[jax.ShapeDtypeStruct((NP, 16), jnp.float32),
                   jax.ShapeDtypeStruct((NP, 16), jnp.float32)],
    )(p2.reshape(2, NP, HD), cnt, h1, Wl2, Wr2, b2.reshape(1, DIM), w3l, w3r)

    a3 = agg_16(src, dst, y3w)[0]
    h3w = pl.pallas_call(
        _tc_layer3,
        grid=(GRID,),
        in_specs=[_spec_16x2(), _spec_rows(16), _spec_rows(16),
                  _spec_full((1, 1))],
        out_specs=_spec_rows(16),
        out_shape=jax.ShapeDtypeStruct((NP, 16), jnp.float32),
    )(a3.reshape(2, NP, 16), cnt, z3w, b3.reshape(1, 1))

    a4 = agg_16(src, dst, h3w)[0]
    out = pl.pallas_call(
        _tc_layer4,
        grid=(GRID,),
        in_specs=[_spec_16x2(), _spec_rows(16), _spec_rows(16),
                  _spec_full((1, 16)), _spec_full((1, 16)),
                  _spec_full((1, 16))],
        out_specs=_spec_rows(16),
        out_shape=jax.ShapeDtypeStruct((NP, 16), jnp.float32),
    )(a4.reshape(2, NP, 16), cnt, h3w, Wl4, Wr4, b4.reshape(1, 16))

    n = x.shape[0]
    return (out[:n], h3w[:n, 0])


# speedup vs baseline: 8.2511x; 1.0141x over previous
import jax
import jax.numpy as jnp
from jax import lax
from jax.experimental import pallas as pl
from jax.experimental.pallas import tpu as pltpu
from jax.experimental.pallas import tpu_sc as plsc

N_NODES = 10000
NP = 10240
DIM = 128
HD = DIM // 2
NC, NS = 2, 16
NW = NC * NS
CH = 128
ROWS_PER_TILE = NP // NS
SLICES_PER_TILE = ROWS_PER_TILE // CH


def _sc_mesh():
    return plsc.VectorSubcoreMesh(
        core_axis_name="c", subcore_axis_name="s",
        num_cores=NC, num_subcores=NS)


NBUF = 8


def _make_agg(e_pad, d, split_features, with_count):
    if split_features:
        chunks_per_worker = e_pad // (NS * CH)
    else:
        chunks_per_worker = e_pad // (NW * CH)
    nsuper = chunks_per_worker // NBUF
    n_out = 2 if with_count else 1

    def body(src_hbm, dst_hbm, x_hbm, *refs):
        if with_count:
            out_hbm, cnt_hbm = refs[0], refs[1]
        else:
            out_hbm = refs[0]
        scratch = refs[n_out:]
        if with_count:
            sidx, didx, rows, z16, ones, acc_s, cnt_s, gsem, ssem, csem = scratch
        else:
            sidx, didx, rows, z16, acc_s, gsem, ssem = scratch

        cid = lax.axis_index("c")
        sid = lax.axis_index("s")
        wid = sid * NC + cid

        def fill(r, _):
            for k in range(d // 16):
                rows[0, r, pl.ds(k * 16, 16)] = jnp.zeros((16,), jnp.float32)
            z16[r] = jnp.zeros((16,), jnp.float32)
            if with_count:
                ones[r] = jnp.ones((16,), jnp.float32)
            return 0
        lax.fori_loop(0, CH, fill, 0)

        for j in range(SLICES_PER_TILE):
            r0 = sid * ROWS_PER_TILE + j * CH
            pltpu.sync_copy(rows.at[0], acc_s.at[pl.ds(r0, CH)])
            if with_count:
                pltpu.sync_copy(z16, cnt_s.at[pl.ds(r0, CH)])
        plsc.subcore_barrier()

        base0 = (sid if split_features else wid) * chunks_per_worker

        def load_idx(s, parity):
            row0 = base0 + s * NBUF
            plane = cid if split_features else 0
            pltpu.sync_copy(src_hbm.at[plane, pl.ds(row0, NBUF)],
                            sidx.at[parity])
            pltpu.sync_copy(dst_hbm.at[pl.ds(row0, NBUF)], didx.at[parity])

        def gather(parity, b):
            return pltpu.make_async_copy(
                x_hbm.at[sidx.at[parity, b]], rows.at[b], gsem.at[b])

        def scatter(parity, b):
            return pltpu.make_async_copy(
                rows.at[b], acc_s.at[didx.at[parity, b]], ssem.at[b])

        def cscatter(parity, b):
            return pltpu.make_async_copy(
                ones, cnt_s.at[didx.at[parity, b]], csem.at[b])

        load_idx(0, 0)
        for b in range(NBUF):
            gather(0, b).start()

        def superstep(s, _):
            p = lax.rem(s, 2)
            np_ = 1 - p

            @pl.when(s < nsuper - 1)
            def _():
                load_idx(s + 1, np_)

            for b in range(NBUF):
                gather(p, b).wait()
                scatter(p, b).start(add=True)
                if with_count:
                    @pl.when(cid == 0)
                    def _():
                        cscatter(p, b).start(add=True)
            for b in range(NBUF):
                scatter(p, b).wait()
                if with_count:
                    @pl.when(cid == 0)
                    def _():
                        cscatter(p, b).wait()

                @pl.when(s < nsuper - 1)
                def _():
                    gather(np_, b).start()
            return 0
        lax.fori_loop(0, nsuper, superstep, 0)
        plsc.subcore_barrier()

        for j in range(SLICES_PER_TILE):
            r0 = sid * ROWS_PER_TILE + j * CH
            pltpu.sync_copy(acc_s.at[pl.ds(r0, CH)],
                            out_hbm.at[pl.ds(cid * NP + r0, CH)])
            if with_count:
                @pl.when(cid == 0)
                def _():
                    pltpu.sync_copy(cnt_s.at[pl.ds(r0, CH)],
                                    cnt_hbm.at[pl.ds(r0, CH)])

    out_type = [jax.ShapeDtypeStruct((NC * NP, d), jnp.float32)]
    scratch = [
        pltpu.VMEM((2, NBUF, CH), jnp.int32),
        pltpu.VMEM((2, NBUF, CH), jnp.int32),
        pltpu.VMEM((NBUF, CH, d), jnp.float32),
        pltpu.VMEM((CH, 16), jnp.float32),
    ]
    if with_count:
        out_type.append(jax.ShapeDtypeStruct((NP, 16), jnp.float32))
        scratch.append(pltpu.VMEM((CH, 16), jnp.float32))
    scratch.append(pltpu.VMEM_SHARED((NP, d), jnp.float32))
    if with_count:
        scratch.append(pltpu.VMEM_SHARED((NP, 16), jnp.float32))
    scratch.append(pltpu.SemaphoreType.DMA((NBUF,)))
    scratch.append(pltpu.SemaphoreType.DMA((NBUF,)))
    if with_count:
        scratch.append(pltpu.SemaphoreType.DMA((NBUF,)))

    return pl.kernel(body, out_type=out_type, mesh=_sc_mesh(),
                     scratch_types=scratch,
                     compiler_params=pltpu.CompilerParams(
                         use_tc_tiling_on_sc=False))


NB = 1024
GRID = NP // NB


def _invd(cnt_ref):
    deg = cnt_ref[:, 0:1]
    return 1.0 / jnp.maximum(deg, 1.0)


def _tc_layer12(p_ref, cnt_ref, h_ref, wl_ref, wr_ref, b_ref, o_ref):
    agg = jnp.concatenate([p_ref[0], p_ref[1]], axis=1) * _invd(cnt_ref)
    h = (jnp.dot(agg, wl_ref[...], preferred_element_type=jnp.float32)
         + jnp.dot(h_ref[...], wr_ref[...], preferred_element_type=jnp.float32)
         + b_ref[...])
    o_ref[...] = jnp.maximum(h, 0.0)


def _tc_layer2b(p_ref, cnt_ref, h_ref, wl_ref, wr_ref, b_ref, w3l_ref, w3r_ref,
                y_ref, z_ref):
    agg = jnp.concatenate([p_ref[0], p_ref[1]], axis=1) * _invd(cnt_ref)
    h2 = (jnp.dot(agg, wl_ref[...], preferred_element_type=jnp.float32)
          + jnp.dot(h_ref[...], wr_ref[...], preferred_element_type=jnp.float32)
          + b_ref[...])
    h2 = jnp.maximum(h2, 0.0)
    y_ref[...] = jnp.dot(h2, w3l_ref[...], preferred_element_type=jnp.float32)
    z_ref[...] = jnp.dot(h2, w3r_ref[...], preferred_element_type=jnp.float32)


def _tc_layer3(a_ref, cnt_ref, z_ref, b_ref, o_ref):
    h3 = ((a_ref[0, :, 0:1] + a_ref[1, :, 0:1]) * _invd(cnt_ref)
          + z_ref[:, 0:1] + b_ref[...])
    cols = lax.broadcasted_iota(jnp.int32, (NB, 16), 1)
    o_ref[...] = jnp.where(cols == 0, h3, 0.0)


def _tc_layer4(a_ref, cnt_ref, h3_ref, wl_ref, wr_ref, b_ref, o_ref):
    a4n = (a_ref[0, :, 0:1] + a_ref[1, :, 0:1]) * _invd(cnt_ref)
    logits = (a4n * wl_ref[...] + h3_ref[:, 0:1] * wr_ref[...] + b_ref[...])
    m = jnp.max(logits, axis=1, keepdims=True)
    sh = logits - m
    lse = jnp.log(jnp.sum(jnp.exp(sh), axis=1, keepdims=True))
    o_ref[...] = sh - lse


def _spec_ph():
    return pl.BlockSpec((2, NB, HD), lambda i: (0, i, 0))


def _spec_16x2():
    return pl.BlockSpec((2, NB, 16), lambda i: (0, i, 0))


def _spec_rows(w):
    return pl.BlockSpec((NB, w), lambda i: (i, 0))


def _spec_full(shape):
    return pl.BlockSpec(shape, lambda i: tuple(0 for _ in shape))


def kernel(x, edge_index, Wl1, Wr1, b1, Wl2, Wr2, b2, Wl3, Wr3, b3,
           Wl4, Wr4, b4):
    x = x.astype(jnp.float32)
    src = edge_index[0].astype(jnp.int32)
    dst = edge_index[1].astype(jnp.int32)
    e = src.shape[0]
    quantum = NW * CH * NBUF
    e_pad = -(-e // quantum) * quantum
    if e_pad != e:
        src = jnp.concatenate([src, jnp.zeros((e_pad - e,), jnp.int32)])
        dst = jnp.concatenate(
            [dst, jnp.full((e_pad - e,), N_NODES, jnp.int32)])
    src = src.reshape(e_pad // CH, CH)
    src = jnp.stack([src, src + NP])
    dst = dst.reshape(e_pad // CH, CH)
    xp = jnp.pad(x, ((0, NP - x.shape[0]), (0, 0)))

    agg_wide_cnt = _make_agg(e_pad, HD, True, True)
    agg_wide = _make_agg(e_pad, HD, True, False)
    agg_16 = _make_agg(e_pad, 16, False, False)

    def halves(v):
        return v.reshape(NP, 2, HD).transpose(1, 0, 2).reshape(2 * NP, HD)

    p1, cnt = agg_wide_cnt(src, dst, halves(xp))
    h1 = pl.pallas_call(
        _tc_layer12,
        grid=(GRID,),
        in_specs=[_spec_ph(), _spec_rows(16), _spec_rows(DIM),
                  _spec_full((DIM, DIM)), _spec_full((DIM, DIM)),
                  _spec_full((1, DIM))],
        out_specs=_spec_rows(DIM),
        out_shape=jax.ShapeDtypeStruct((NP, DIM), jnp.float32),
    )(p1.reshape(2, NP, HD), cnt, xp, Wl1, Wr1, b1.reshape(1, DIM))

    p2 = agg_wide(src, dst, halves(h1))[0]
    w3l = jnp.pad(Wl3, ((0, 0), (0, 15)))
    w3r = jnp.pad(Wr3, ((0, 0), (0, 15)))
    y3w, z3w = pl.pallas_call(
        _tc_layer2b,
        grid=(GRID,),
        in_specs=[_spec_ph(), _spec_rows(16), _spec_rows(DIM),
                  _spec_full((DIM, DIM)), _spec_full((DIM, DIM)),
                  _spec_full((1, DIM)), _spec_full((DIM, 16)),
                  _spec_full((DIM, 16))],
        out_specs=[_spec_rows(16), _spec_rows(16)],
        out_shape=[jax.ShapeDtypeStruct((NP, 16), jnp.float32),
                   jax.ShapeDtypeStruct((NP, 16), jnp.float32)],
    )(p2.reshape(2, NP, HD), cnt, h1, Wl2, Wr2, b2.reshape(1, DIM), w3l, w3r)

    a3 = agg_16(src, dst, y3w)[0]
    h3w = pl.pallas_call(
        _tc_layer3,
        grid=(GRID,),
        in_specs=[_spec_16x2(), _spec_rows(16), _spec_rows(16),
                  _spec_full((1, 1))],
        out_specs=_spec_rows(16),
        out_shape=jax.ShapeDtypeStruct((NP, 16), jnp.float32),
    )(a3.reshape(2, NP, 16), cnt, z3w, b3.reshape(1, 1))

    a4 = agg_16(src, dst, h3w)[0]
    out = pl.pallas_call(
        _tc_layer4,
        grid=(GRID,),
        in_specs=[_spec_16x2(), _spec_rows(16), _spec_rows(16),
                  _spec_full((1, 16)), _spec_full((1, 16)),
                  _spec_full((1, 16))],
        out_specs=_spec_rows(16),
        out_shape=jax.ShapeDtypeStruct((NP, 16), jnp.float32),
    )(a4.reshape(2, NP, 16), cnt, h3w, Wl4, Wr4, b4.reshape(1, 16))

    n = x.shape[0]
    return (out[:n], h3w[:n, 0])
